# baseline (speedup 1.0000x reference)
import jax
import jax.numpy as jnp
from jax import lax
from jax.experimental import pallas as pl
from jax.experimental.pallas import tpu as pltpu
from jax.experimental.pallas import tpu_sc as plsc

_NV = 10000
_NF = 10000
_E = 160000
_D = 128

_NC = 2
_NS = 16
_NW = _NC * _NS
_L = 16

_EW = _E // _NW
_C = 50
_K = _EW // _C
_STR = _NV // _NS
_ZJ = _STR // _C
_ZT = _STR - _ZJ * _C


def _mesh():
    return plsc.VectorSubcoreMesh(
        core_axis_name="c", subcore_axis_name="s",
        num_cores=_NC, num_subcores=_NS)


def _params():
    return pltpu.CompilerParams(use_tc_tiling_on_sc=False,
                                needs_layout_passes=False)


def _zero_buf(buf):
    zv = jnp.zeros((_L,), jnp.float32)

    def zrow(r, carry):
        for k in range(_D // _L):
            buf[r, pl.ds(k * _L, _L)] = zv
        return carry

    lax.fori_loop(0, _C, zrow, 0)


def _zero_acc_stripe(acc, zbuf, sid):
    _zero_buf(zbuf)
    for j in range(_ZJ):
        pltpu.sync_copy(zbuf, acc.at[pl.ds(sid * _STR + j * _C, _C)])
    pltpu.sync_copy(zbuf.at[pl.ds(0, _ZT)],
                    acc.at[pl.ds(sid * _STR + _ZJ * _C, _ZT)])


def _flush_acc_stripe(acc, out_hbm, cid, sid):
    for j in range(_ZJ):
        sl = pl.ds(sid * _STR + j * _C, _C)
        pltpu.sync_copy(acc.at[sl], out_hbm.at[cid, sl])
    sl = pl.ds(sid * _STR + _ZJ * _C, _ZT)
    pltpu.sync_copy(acc.at[sl], out_hbm.at[cid, sl])


def _relu_add(dst, src):
    def erow(r, carry):
        for j in range(_D // (2 * _L)):
            v = src[r, pl.ds(j * 2 * _L, 2 * _L)]
            a, b = plsc.unpack(v, format=plsc.PackFormat.INTERLEAVED)
            sla = pl.ds(j * 2 * _L, _L)
            slb = pl.ds(j * 2 * _L + _L, _L)
            dst[r, sla] = jnp.maximum(dst[r, sla] + a, 0.0)
            dst[r, slb] = jnp.maximum(dst[r, slb] + b, 0.0)
        return carry

    lax.fori_loop(0, _C, erow, 0)


def _sc_segsum_body(data_hbm, idx_hbm, out_hbm, idxv,
                    r0, r1, r2, r3, l0, l1, l2, l3, s0, s1, s2, s3, acc):
    cid = lax.axis_index("c")
    sid = lax.axis_index("s")
    wid = cid * _NS + sid
    base = wid * _EW
    rows = (r0, r1, r2, r3)
    lsem = (l0, l1, l2, l3)
    ssem = (s0, s1, s2, s3)
    pltpu.sync_copy(idx_hbm.at[wid], idxv)
    _zero_acc_stripe(acc, r0, sid)
    for b in range(2):
        pltpu.async_copy(data_hbm.at[pl.ds(base + b * _C, _C)],
                         rows[b], lsem[b])
    plsc.subcore_barrier()

    def outer(c2, carry):
        for b in range(4):
            c = c2 * 4 + b
            pltpu.make_async_copy(data_hbm.at[pl.ds(base + c * _C, _C)],
                                  rows[b], lsem[b]).wait()
            pltpu.async_copy(rows[b], acc.at[idxv.at[c]], ssem[b], add=True)
            b2 = (b + 2) % 4

            def _wait_prev(b2=b2):
                pltpu.make_async_copy(
                    rows[b2], acc.at[idxv.at[c - 2]], ssem[b2]).wait()

            def _issue(b2=b2):
                pltpu.async_copy(
                    data_hbm.at[pl.ds(base + (c + 2) * _C, _C)],
                    rows[b2], lsem[b2])

            if b < 2:
                pl.when(c2 > 0)(_wait_prev)
                _issue()
            else:
                _wait_prev()
                pl.when(c2 < _K // 4 - 1)(_issue)
        return carry

    lax.fori_loop(0, _K // 4, outer, 0)
    for b in (2, 3):
        pltpu.make_async_copy(
            rows[b], acc.at[idxv.at[_K - 4 + b]], ssem[b]).wait()
    plsc.subcore_barrier()
    _flush_acc_stripe(acc, out_hbm, cid, sid)


def _sc_segsum(data, idx3):
    return pl.kernel(
        _sc_segsum_body,
        out_type=jax.ShapeDtypeStruct((_NC, _NV, _D), jnp.float32),
        mesh=_mesh(),
        compiler_params=_params(),
        scratch_types=[
            pltpu.VMEM((_K, _C), jnp.int32),
            pltpu.VMEM((_C, _D), jnp.float32),
            pltpu.VMEM((_C, _D), jnp.float32),
            pltpu.VMEM((_C, _D), jnp.float32),
            pltpu.VMEM((_C, _D), jnp.float32),
            pltpu.SemaphoreType.DMA,
            pltpu.SemaphoreType.DMA,
            pltpu.SemaphoreType.DMA,
            pltpu.SemaphoreType.DMA,
            pltpu.SemaphoreType.DMA,
            pltpu.SemaphoreType.DMA,
            pltpu.SemaphoreType.DMA,
            pltpu.SemaphoreType.DMA,
            pltpu.VMEM_SHARED((_NV, _D), jnp.float32),
        ],
    )(data, idx3)


def _sc_grs_body(a_hbm, b_hbm, gidx_hbm, sidx_hbm, m_hbm, out_hbm,
                 gidxv, sidxv, g0, g1, g2, g3, bb0, bb1,
                 gs0, gs1, gs2, gs3, ms0, ms1, ms2, ms3,
                 ss0, ss1, ss2, ss3, bs0, bs1, acc):
    cid = lax.axis_index("c")
    sid = lax.axis_index("s")
    wid = cid * _NS + sid
    base = wid * _EW
    g = (g0, g1, g2, g3)
    bb = (bb0, bb1)
    gsem = (gs0, gs1, gs2, gs3)
    msem = (ms0, ms1, ms2, ms3)
    ssem = (ss0, ss1, ss2, ss3)
    bsem = (bs0, bs1)
    pltpu.sync_copy(gidx_hbm.at[wid], gidxv)
    pltpu.sync_copy(sidx_hbm.at[wid], sidxv)
    _zero_acc_stripe(acc, g0, sid)
    for b in range(2):
        pltpu.async_copy(a_hbm.at[gidxv.at[b]], g[b], gsem[b])
        pltpu.async_copy(b_hbm.at[pl.ds(base + b * _C, _C)], bb[b], bsem[b])
    plsc.subcore_barrier()

    def outer(c2, carry):
        for b in range(4):
            bi = b % 2
            c = c2 * 4 + b
            pltpu.make_async_copy(a_hbm.at[gidxv.at[c]], g[b],
                                  gsem[b]).wait()
            pltpu.make_async_copy(b_hbm.at[pl.ds(base + c * _C, _C)],
                                  bb[bi], bsem[bi]).wait()
            _relu_add(g[b], bb[bi])

            def _issue_b(bi=bi):
                pltpu.async_copy(b_hbm.at[pl.ds(base + (c + 2) * _C, _C)],
                                 bb[bi], bsem[bi])

            if b < 2:
                _issue_b()
            else:
                pl.when(c2 < _K // 4 - 1)(_issue_b)
            pltpu.async_copy(g[b], m_hbm.at[pl.ds(base + c * _C, _C)],
                             msem[b])
            pltpu.async_copy(g[b], acc.at[sidxv.at[c]], ssem[b], add=True)
            b2 = (b + 2) % 4

            def _wait_prev(b2=b2):
                pltpu.make_async_copy(
                    g[b2], m_hbm.at[pl.ds(base + (c - 2) * _C, _C)],
                    msem[b2]).wait()
                pltpu.make_async_copy(
                    g[b2], acc.at[sidxv.at[c - 2]], ssem[b2]).wait()

            def _issue_g(b2=b2):
                pltpu.async_copy(a_hbm.at[gidxv.at[c + 2]], g[b2], gsem[b2])

            if b < 2:
                pl.when(c2 > 0)(_wait_prev)
                _issue_g()
            else:
                _wait_prev()
                pl.when(c2 < _K // 4 - 1)(_issue_g)
        return carry

    lax.fori_loop(0, _K // 4, outer, 0)
    for b in (2, 3):
        c = _K - 4 + b
        pltpu.make_async_copy(
            g[b], m_hbm.at[pl.ds(base + c * _C, _C)], msem[b]).wait()
        pltpu.make_async_copy(g[b], acc.at[sidxv.at[c]], ssem[b]).wait()
    plsc.subcore_barrier()
    _flush_acc_stripe(acc, out_hbm, cid, sid)


def _sc_gather_relu_scatter(a, b, gidx3, sidx3):
    return pl.kernel(
        _sc_grs_body,
        out_type=(jax.ShapeDtypeStruct((_E, _D), jnp.float32),
                  jax.ShapeDtypeStruct((_NC, _NF, _D), jnp.float32)),
        mesh=_mesh(),
        compiler_params=_params(),
        scratch_types=(
            [pltpu.VMEM((_K, _C), jnp.int32)] * 2
            + [pltpu.VMEM((_C, _D), jnp.float32)] * 4
            + [pltpu.VMEM((_C, _D), jnp.bfloat16)] * 2
            + [pltpu.SemaphoreType.DMA] * 14
            + [pltpu.VMEM_SHARED((_NF, _D), jnp.float32)]
        ),
    )(a, b, gidx3, sidx3)


def _sc_gr_body(a_hbm, b_hbm, gidx_hbm, m_hbm,
                gidxv, g0, g1, g2, g3, bb0, bb1,
                gs0, gs1, gs2, gs3, ms0, ms1, ms2, ms3,
                bs0, bs1, av_sh):
    cid = lax.axis_index("c")
    sid = lax.axis_index("s")
    wid = cid * _NS + sid
    base = wid * _EW
    g = (g0, g1, g2, g3)
    bb = (bb0, bb1)
    gsem = (gs0, gs1, gs2, gs3)
    msem = (ms0, ms1, ms2, ms3)
    bsem = (bs0, bs1)
    pltpu.sync_copy(gidx_hbm.at[wid], gidxv)
    for j in range(_ZJ):
        sl = pl.ds(sid * _STR + j * _C, _C)
        pltpu.sync_copy(a_hbm.at[sl], av_sh.at[sl])
    sl = pl.ds(sid * _STR + _ZJ * _C, _ZT)
    pltpu.sync_copy(a_hbm.at[sl], av_sh.at[sl])
    for b in range(2):
        pltpu.async_copy(b_hbm.at[pl.ds(base + b * _C, _C)], bb[b], bsem[b])
    plsc.subcore_barrier()
    for b in range(2):
        pltpu.async_copy(av_sh.at[gidxv.at[b]], g[b], gsem[b])

    def outer(c2, carry):
        for b in range(4):
            bi = b % 2
            c = c2 * 4 + b
            pltpu.make_async_copy(av_sh.at[gidxv.at[c]], g[b],
                                  gsem[b]).wait()
            pltpu.make_async_copy(b_hbm.at[pl.ds(base + c * _C, _C)],
                                  bb[bi], bsem[bi]).wait()
            _relu_add(g[b], bb[bi])

            def _issue_b(bi=bi):
                pltpu.async_copy(b_hbm.at[pl.ds(base + (c + 2) * _C, _C)],
                                 bb[bi], bsem[bi])

            if b < 2:
                _issue_b()
            else:
                pl.when(c2 < _K // 4 - 1)(_issue_b)
            pltpu.async_copy(g[b], m_hbm.at[pl.ds(base + c * _C, _C)],
                             msem[b])
            b2 = (b + 2) % 4

            def _wait_prev(b2=b2):
                pltpu.make_async_copy(
                    g[b2], m_hbm.at[pl.ds(base + (c - 2) * _C, _C)],
                    msem[b2]).wait()

            def _issue_g(b2=b2):
                pltpu.async_copy(av_sh.at[gidxv.at[c + 2]], g[b2], gsem[b2])

            if b < 2:
                pl.when(c2 > 0)(_wait_prev)
                _issue_g()
            else:
                _wait_prev()
                pl.when(c2 < _K // 4 - 1)(_issue_g)
        return carry

    lax.fori_loop(0, _K // 4, outer, 0)
    for b in (2, 3):
        c = _K - 4 + b
        pltpu.make_async_copy(
            g[b], m_hbm.at[pl.ds(base + c * _C, _C)], msem[b]).wait()


def _sc_gather_relu(a, b, gidx3):
    return pl.kernel(
        _sc_gr_body,
        out_type=jax.ShapeDtypeStruct((_E, _D), jnp.float32),
        mesh=_mesh(),
        compiler_params=_params(),
        scratch_types=(
            [pltpu.VMEM((_K, _C), jnp.int32)]
            + [pltpu.VMEM((_C, _D), jnp.float32)] * 4
            + [pltpu.VMEM((_C, _D), jnp.bfloat16)] * 2
            + [pltpu.SemaphoreType.DMA] * 10
            + [pltpu.VMEM_SHARED((_NF, _D), jnp.float32)]
        ),
    )(a, b, gidx3)


_EB = 800


def _interleave_cols(w):
    perm = jnp.arange(_D).reshape(_D // 32, 2, 16).transpose(0, 2, 1).reshape(_D)
    return w[:, perm]


def _tc_cat_body(x_ref, y_ref, w_ref, o_ref):
    h = jnp.concatenate((x_ref[...], y_ref[...]), axis=-1)
    r = jnp.dot(h, w_ref[...], preferred_element_type=jnp.float32)
    o_ref[...] = r.astype(jnp.bfloat16)


def _tc_edge_cat(x, y, wcat, eb=_EB):
    return pl.pallas_call(
        _tc_cat_body,
        grid=(_E // eb,),
        in_specs=[
            pl.BlockSpec((eb, _D), lambda i: (i, 0)),
            pl.BlockSpec((eb, _D), lambda i: (i, 0)),
            pl.BlockSpec((2 * _D, _D), lambda i: (0, 0)),
        ],
        out_specs=pl.BlockSpec((eb, _D), lambda i: (i, 0)),
        out_shape=jax.ShapeDtypeStruct((_E, _D), jnp.bfloat16),
    )(x, y, _interleave_cols(wcat))


_NB = 2000


def _tc_node_body(s0_ref, s1_ref, x_ref, w1_ref, w2_ref, b_ref,
                  a_ref, nx_ref):
    s = s0_ref[0] + s1_ref[0]
    a_ref[...] = (
        jnp.dot(x_ref[...], w1_ref[...], preferred_element_type=jnp.float32)
        + jnp.dot(s, w2_ref[...], preferred_element_type=jnp.float32)
        + b_ref[...])
    nx_ref[...] = s + x_ref[...]


def _tc_node_combo(sp, x, w1, w2, b):
    n = x.shape[0]
    return pl.pallas_call(
        _tc_node_body,
        grid=(n // _NB,),
        in_specs=[
            pl.BlockSpec((1, _NB, _D), lambda i: (0, i, 0)),
            pl.BlockSpec((1, _NB, _D), lambda i: (1, i, 0)),
            pl.BlockSpec((_NB, _D), lambda i: (i, 0)),
            pl.BlockSpec((_D, _D), lambda i: (0, 0)),
            pl.BlockSpec((_D, _D), lambda i: (0, 0)),
            pl.BlockSpec((1, _D), lambda i: (0, 0)),
        ],
        out_specs=[
            pl.BlockSpec((_NB, _D), lambda i: (i, 0)),
            pl.BlockSpec((_NB, _D), lambda i: (i, 0)),
        ],
        out_shape=[
            jax.ShapeDtypeStruct((n, _D), jnp.float32),
            jax.ShapeDtypeStruct((n, _D), jnp.float32),
        ],
    )(sp, sp, x, w1, w2, b)


def kernel(variable, factor, edge_attr, prev_m_f_to_v, v_to_f, f_to_v,
           W_v, b_v, W_f, b_f):
    gidx = v_to_f.reshape(_NW, _K, _C)
    sidx = f_to_v.reshape(_NW, _K, _C)
    Wv1, Wv2, Wv3 = W_v[:_D], W_v[_D:2 * _D], W_v[2 * _D:]
    Wf1, Wf2, Wf3 = W_f[:_D], W_f[_D:2 * _D], W_f[2 * _D:]

    s1p = _sc_segsum(prev_m_f_to_v, gidx)
    b1 = _tc_edge_cat(edge_attr, prev_m_f_to_v,
                      jnp.concatenate((Wv3, -Wv2), axis=0))
    a_v, new_variable = _tc_node_combo(s1p, variable, Wv1, Wv2,
                                       b_v.reshape(1, _D))
    m_v_to_f, s2p = _sc_gather_relu_scatter(a_v, b1, gidx, sidx)
    a_f, new_factor = _tc_node_combo(s2p, factor, Wf1, Wf2,
                                     b_f.reshape(1, _D))
    p2 = _tc_edge_cat(edge_attr, m_v_to_f,
                      jnp.concatenate((Wf3, -Wf2), axis=0))
    m_f_to_v = _sc_gather_relu(a_f, p2, sidx)
    return (m_f_to_v, new_factor, new_variable)


# speedup vs baseline: 1.5552x; 1.5552x over previous
import jax
import jax.numpy as jnp
from jax import lax
from jax.experimental import pallas as pl
from jax.experimental.pallas import tpu as pltpu
from jax.experimental.pallas import tpu_sc as plsc

_NV = 10000
_NF = 10000
_E = 160000
_D = 128

_NC = 2
_NS = 16
_NW = _NC * _NS
_L = 16

_EW = _E // _NW
_C = 50
_K = _EW // _C
_STR = _NV // _NS
_ZJ = _STR // _C
_ZT = _STR - _ZJ * _C


def _mesh():
    return plsc.VectorSubcoreMesh(
        core_axis_name="c", subcore_axis_name="s",
        num_cores=_NC, num_subcores=_NS)


def _params():
    return pltpu.CompilerParams(use_tc_tiling_on_sc=False)


def _zero_buf(buf):
    zv = jnp.zeros((_L,), jnp.float32)

    def zrow(r, carry):
        for k in range(_D // _L):
            buf[r, pl.ds(k * _L, _L)] = zv
        return carry

    lax.fori_loop(0, _C, zrow, 0)


def _zero_acc_stripe(acc, zbuf, sid):
    _zero_buf(zbuf)
    for j in range(_ZJ):
        pltpu.sync_copy(zbuf, acc.at[pl.ds(sid * _STR + j * _C, _C)])
    pltpu.sync_copy(zbuf.at[pl.ds(0, _ZT)],
                    acc.at[pl.ds(sid * _STR + _ZJ * _C, _ZT)])


def _flush_acc_stripe(acc, out_hbm, cid, sid):
    for j in range(_ZJ):
        sl = pl.ds(sid * _STR + j * _C, _C)
        pltpu.sync_copy(acc.at[sl], out_hbm.at[cid, sl])
    sl = pl.ds(sid * _STR + _ZJ * _C, _ZT)
    pltpu.sync_copy(acc.at[sl], out_hbm.at[cid, sl])


def _relu_add(dst, src):
    def erow(r, carry):
        for k in range(_D // _L):
            sl = pl.ds(k * _L, _L)
            dst[r, sl] = jnp.maximum(dst[r, sl] + src[r, sl], 0.0)
        return carry

    lax.fori_loop(0, _C, erow, 0)


def _sc_segsum_body(data_hbm, idx_hbm, out_hbm, idxv,
                    r0, r1, r2, r3, l0, l1, l2, l3, s0, s1, s2, s3, acc):
    cid = lax.axis_index("c")
    sid = lax.axis_index("s")
    wid = cid * _NS + sid
    base = wid * _EW
    rows = (r0, r1, r2, r3)
    lsem = (l0, l1, l2, l3)
    ssem = (s0, s1, s2, s3)
    pltpu.sync_copy(idx_hbm.at[wid], idxv)
    _zero_acc_stripe(acc, r0, sid)
    for b in range(2):
        pltpu.async_copy(data_hbm.at[pl.ds(base + b * _C, _C)],
                         rows[b], lsem[b])
    plsc.subcore_barrier()

    def outer(c2, carry):
        for b in range(4):
            c = c2 * 4 + b
            pltpu.make_async_copy(data_hbm.at[pl.ds(base + c * _C, _C)],
                                  rows[b], lsem[b]).wait()
            pltpu.async_copy(rows[b], acc.at[idxv.at[c]], ssem[b], add=True)
            b2 = (b + 2) % 4

            def _wait_prev(b2=b2):
                pltpu.make_async_copy(
                    rows[b2], acc.at[idxv.at[c - 2]], ssem[b2]).wait()

            def _issue(b2=b2):
                pltpu.async_copy(
                    data_hbm.at[pl.ds(base + (c + 2) * _C, _C)],
                    rows[b2], lsem[b2])

            if b < 2:
                pl.when(c2 > 0)(_wait_prev)
                _issue()
            else:
                _wait_prev()
                pl.when(c2 < _K // 4 - 1)(_issue)
        return carry

    lax.fori_loop(0, _K // 4, outer, 0)
    for b in (2, 3):
        pltpu.make_async_copy(
            rows[b], acc.at[idxv.at[_K - 4 + b]], ssem[b]).wait()
    plsc.subcore_barrier()
    _flush_acc_stripe(acc, out_hbm, cid, sid)


def _sc_segsum(data, idx3):
    return pl.kernel(
        _sc_segsum_body,
        out_type=jax.ShapeDtypeStruct((_NC, _NV, _D), jnp.float32),
        mesh=_mesh(),
        compiler_params=_params(),
        scratch_types=[
            pltpu.VMEM((_K, _C), jnp.int32),
            pltpu.VMEM((_C, _D), jnp.float32),
            pltpu.VMEM((_C, _D), jnp.float32),
            pltpu.VMEM((_C, _D), jnp.float32),
            pltpu.VMEM((_C, _D), jnp.float32),
            pltpu.SemaphoreType.DMA,
            pltpu.SemaphoreType.DMA,
            pltpu.SemaphoreType.DMA,
            pltpu.SemaphoreType.DMA,
            pltpu.SemaphoreType.DMA,
            pltpu.SemaphoreType.DMA,
            pltpu.SemaphoreType.DMA,
            pltpu.SemaphoreType.DMA,
            pltpu.VMEM_SHARED((_NV, _D), jnp.float32),
        ],
    )(data, idx3)


def _sc_grs_body(a_hbm, b_hbm, gidx_hbm, sidx_hbm, m_hbm, out_hbm,
                 gidxv, sidxv, g0, g1, g2, g3, bb0, bb1,
                 gs0, gs1, gs2, gs3, ms0, ms1, ms2, ms3,
                 ss0, ss1, ss2, ss3, bs0, bs1, acc):
    cid = lax.axis_index("c")
    sid = lax.axis_index("s")
    wid = cid * _NS + sid
    base = wid * _EW
    g = (g0, g1, g2, g3)
    bb = (bb0, bb1)
    gsem = (gs0, gs1, gs2, gs3)
    msem = (ms0, ms1, ms2, ms3)
    ssem = (ss0, ss1, ss2, ss3)
    bsem = (bs0, bs1)
    pltpu.sync_copy(gidx_hbm.at[wid], gidxv)
    pltpu.sync_copy(sidx_hbm.at[wid], sidxv)
    _zero_acc_stripe(acc, g0, sid)
    for b in range(2):
        pltpu.async_copy(a_hbm.at[gidxv.at[b]], g[b], gsem[b])
        pltpu.async_copy(b_hbm.at[pl.ds(base + b * _C, _C)], bb[b], bsem[b])
    plsc.subcore_barrier()

    def outer(c2, carry):
        for b in range(4):
            bi = b % 2
            c = c2 * 4 + b
            pltpu.make_async_copy(a_hbm.at[gidxv.at[c]], g[b],
                                  gsem[b]).wait()
            pltpu.make_async_copy(b_hbm.at[pl.ds(base + c * _C, _C)],
                                  bb[bi], bsem[bi]).wait()
            _relu_add(g[b], bb[bi])

            def _issue_b(bi=bi):
                pltpu.async_copy(b_hbm.at[pl.ds(base + (c + 2) * _C, _C)],
                                 bb[bi], bsem[bi])

            if b < 2:
                _issue_b()
            else:
                pl.when(c2 < _K // 4 - 1)(_issue_b)
            pltpu.async_copy(g[b], m_hbm.at[pl.ds(base + c * _C, _C)],
                             msem[b])
            pltpu.async_copy(g[b], acc.at[sidxv.at[c]], ssem[b], add=True)
            b2 = (b + 2) % 4

            def _wait_prev(b2=b2):
                pltpu.make_async_copy(
                    g[b2], m_hbm.at[pl.ds(base + (c - 2) * _C, _C)],
                    msem[b2]).wait()
                pltpu.make_async_copy(
                    g[b2], acc.at[sidxv.at[c - 2]], ssem[b2]).wait()

            def _issue_g(b2=b2):
                pltpu.async_copy(a_hbm.at[gidxv.at[c + 2]], g[b2], gsem[b2])

            if b < 2:
                pl.when(c2 > 0)(_wait_prev)
                _issue_g()
            else:
                _wait_prev()
                pl.when(c2 < _K // 4 - 1)(_issue_g)
        return carry

    lax.fori_loop(0, _K // 4, outer, 0)
    for b in (2, 3):
        c = _K - 4 + b
        pltpu.make_async_copy(
            g[b], m_hbm.at[pl.ds(base + c * _C, _C)], msem[b]).wait()
        pltpu.make_async_copy(g[b], acc.at[sidxv.at[c]], ssem[b]).wait()
    plsc.subcore_barrier()
    _flush_acc_stripe(acc, out_hbm, cid, sid)


def _sc_gather_relu_scatter(a, b, gidx3, sidx3):
    return pl.kernel(
        _sc_grs_body,
        out_type=(jax.ShapeDtypeStruct((_E, _D), jnp.float32),
                  jax.ShapeDtypeStruct((_NC, _NF, _D), jnp.float32)),
        mesh=_mesh(),
        compiler_params=_params(),
        scratch_types=(
            [pltpu.VMEM((_K, _C), jnp.int32)] * 2
            + [pltpu.VMEM((_C, _D), jnp.float32)] * 6
            + [pltpu.SemaphoreType.DMA] * 14
            + [pltpu.VMEM_SHARED((_NF, _D), jnp.float32)]
        ),
    )(a, b, gidx3, sidx3)


def _sc_gr_body(a_hbm, b_hbm, gidx_hbm, m_hbm,
                gidxv, g0, g1, g2, g3, bb0, bb1,
                gs0, gs1, gs2, gs3, ms0, ms1, ms2, ms3,
                bs0, bs1, av_sh):
    cid = lax.axis_index("c")
    sid = lax.axis_index("s")
    wid = cid * _NS + sid
    base = wid * _EW
    g = (g0, g1, g2, g3)
    bb = (bb0, bb1)
    gsem = (gs0, gs1, gs2, gs3)
    msem = (ms0, ms1, ms2, ms3)
    bsem = (bs0, bs1)
    pltpu.sync_copy(gidx_hbm.at[wid], gidxv)
    for j in range(_ZJ):
        sl = pl.ds(sid * _STR + j * _C, _C)
        pltpu.sync_copy(a_hbm.at[sl], av_sh.at[sl])
    sl = pl.ds(sid * _STR + _ZJ * _C, _ZT)
    pltpu.sync_copy(a_hbm.at[sl], av_sh.at[sl])
    for b in range(2):
        pltpu.async_copy(b_hbm.at[pl.ds(base + b * _C, _C)], bb[b], bsem[b])
    plsc.subcore_barrier()
    for b in range(2):
        pltpu.async_copy(av_sh.at[gidxv.at[b]], g[b], gsem[b])

    def outer(c2, carry):
        for b in range(4):
            bi = b % 2
            c = c2 * 4 + b
            pltpu.make_async_copy(av_sh.at[gidxv.at[c]], g[b],
                                  gsem[b]).wait()
            pltpu.make_async_copy(b_hbm.at[pl.ds(base + c * _C, _C)],
                                  bb[bi], bsem[bi]).wait()
            _relu_add(g[b], bb[bi])

            def _issue_b(bi=bi):
                pltpu.async_copy(b_hbm.at[pl.ds(base + (c + 2) * _C, _C)],
                                 bb[bi], bsem[bi])

            if b < 2:
                _issue_b()
            else:
                pl.when(c2 < _K // 4 - 1)(_issue_b)
            pltpu.async_copy(g[b], m_hbm.at[pl.ds(base + c * _C, _C)],
                             msem[b])
            b2 = (b + 2) % 4

            def _wait_prev(b2=b2):
                pltpu.make_async_copy(
                    g[b2], m_hbm.at[pl.ds(base + (c - 2) * _C, _C)],
                    msem[b2]).wait()

            def _issue_g(b2=b2):
                pltpu.async_copy(av_sh.at[gidxv.at[c + 2]], g[b2], gsem[b2])

            if b < 2:
                pl.when(c2 > 0)(_wait_prev)
                _issue_g()
            else:
                _wait_prev()
                pl.when(c2 < _K // 4 - 1)(_issue_g)
        return carry

    lax.fori_loop(0, _K // 4, outer, 0)
    for b in (2, 3):
        c = _K - 4 + b
        pltpu.make_async_copy(
            g[b], m_hbm.at[pl.ds(base + c * _C, _C)], msem[b]).wait()


def _sc_gather_relu(a, b, gidx3):
    return pl.kernel(
        _sc_gr_body,
        out_type=jax.ShapeDtypeStruct((_E, _D), jnp.float32),
        mesh=_mesh(),
        compiler_params=_params(),
        scratch_types=(
            [pltpu.VMEM((_K, _C), jnp.int32)]
            + [pltpu.VMEM((_C, _D), jnp.float32)] * 6
            + [pltpu.SemaphoreType.DMA] * 10
            + [pltpu.VMEM_SHARED((_NF, _D), jnp.float32)]
        ),
    )(a, b, gidx3)


_EB = 800


def _tc_cat_body(x_ref, y_ref, w_ref, o_ref):
    h = jnp.concatenate((x_ref[...], y_ref[...]), axis=-1)
    o_ref[...] = jnp.dot(h, w_ref[...], preferred_element_type=jnp.float32)


def _tc_edge_cat(x, y, wcat, eb=_EB):
    return pl.pallas_call(
        _tc_cat_body,
        grid=(_E // eb,),
        in_specs=[
            pl.BlockSpec((eb, _D), lambda i: (i, 0)),
            pl.BlockSpec((eb, _D), lambda i: (i, 0)),
            pl.BlockSpec((2 * _D, _D), lambda i: (0, 0)),
        ],
        out_specs=pl.BlockSpec((eb, _D), lambda i: (i, 0)),
        out_shape=jax.ShapeDtypeStruct((_E, _D), jnp.float32),
    )(x, y, wcat)


_NB = 2000


def _tc_node_body(s0_ref, s1_ref, x_ref, w1_ref, w2_ref, b_ref,
                  a_ref, nx_ref):
    s = s0_ref[0] + s1_ref[0]
    a_ref[...] = (
        jnp.dot(x_ref[...], w1_ref[...], preferred_element_type=jnp.float32)
        + jnp.dot(s, w2_ref[...], preferred_element_type=jnp.float32)
        + b_ref[...])
    nx_ref[...] = s + x_ref[...]


def _tc_node_combo(sp, x, w1, w2, b):
    n = x.shape[0]
    return pl.pallas_call(
        _tc_node_body,
        grid=(n // _NB,),
        in_specs=[
            pl.BlockSpec((1, _NB, _D), lambda i: (0, i, 0)),
            pl.BlockSpec((1, _NB, _D), lambda i: (1, i, 0)),
            pl.BlockSpec((_NB, _D), lambda i: (i, 0)),
            pl.BlockSpec((_D, _D), lambda i: (0, 0)),
            pl.BlockSpec((_D, _D), lambda i: (0, 0)),
            pl.BlockSpec((1, _D), lambda i: (0, 0)),
        ],
        out_specs=[
            pl.BlockSpec((_NB, _D), lambda i: (i, 0)),
            pl.BlockSpec((_NB, _D), lambda i: (i, 0)),
        ],
        out_shape=[
            jax.ShapeDtypeStruct((n, _D), jnp.float32),
            jax.ShapeDtypeStruct((n, _D), jnp.float32),
        ],
    )(sp, sp, x, w1, w2, b)


def kernel(variable, factor, edge_attr, prev_m_f_to_v, v_to_f, f_to_v,
           W_v, b_v, W_f, b_f):
    gidx = v_to_f.reshape(_NW, _K, _C)
    sidx = f_to_v.reshape(_NW, _K, _C)
    Wv1, Wv2, Wv3 = W_v[:_D], W_v[_D:2 * _D], W_v[2 * _D:]
    Wf1, Wf2, Wf3 = W_f[:_D], W_f[_D:2 * _D], W_f[2 * _D:]

    s1p = _sc_segsum(prev_m_f_to_v, gidx)
    b1 = _tc_edge_cat(edge_attr, prev_m_f_to_v,
                      jnp.concatenate((Wv3, -Wv2), axis=0))
    a_v, new_variable = _tc_node_combo(s1p, variable, Wv1, Wv2,
                                       b_v.reshape(1, _D))
    m_v_to_f, s2p = _sc_gather_relu_scatter(a_v, b1, gidx, sidx)
    a_f, new_factor = _tc_node_combo(s2p, factor, Wf1, Wf2,
                                     b_f.reshape(1, _D))
    p2 = _tc_edge_cat(edge_attr, m_v_to_f,
                      jnp.concatenate((Wf3, -Wf2), axis=0))
    m_f_to_v = _sc_gather_relu(a_f, p2, sidx)
    return (m_f_to_v, new_factor, new_variable)


# speedup vs baseline: 2.0858x; 1.3412x over previous
import jax
import jax.numpy as jnp
from jax import lax
from jax.experimental import pallas as pl
from jax.experimental.pallas import tpu as pltpu
from jax.experimental.pallas import tpu_sc as plsc

_NV = 10000
_NF = 10000
_E = 160000
_D = 128

_NC = 2
_NS = 16
_NW = _NC * _NS
_L = 16

_EW = _E // _NW
_C = 50
_K = _EW // _C
_STR = _NV // _NS
_ZJ = _STR // _C
_ZT = _STR - _ZJ * _C


def _mesh():
    return plsc.VectorSubcoreMesh(
        core_axis_name="c", subcore_axis_name="s",
        num_cores=_NC, num_subcores=_NS)


def _params():
    return pltpu.CompilerParams(use_tc_tiling_on_sc=False)


def _zero_buf(buf):
    zv = jnp.zeros((_L,), jnp.float32)

    def zrow(r, carry):
        for k in range(_D // _L):
            buf[r, pl.ds(k * _L, _L)] = zv
        return carry

    lax.fori_loop(0, _C, zrow, 0)


def _zero_acc_stripe(acc, zbuf, sid):
    _zero_buf(zbuf)
    for j in range(_ZJ):
        pltpu.sync_copy(zbuf, acc.at[pl.ds(sid * _STR + j * _C, _C)])
    pltpu.sync_copy(zbuf.at[pl.ds(0, _ZT)],
                    acc.at[pl.ds(sid * _STR + _ZJ * _C, _ZT)])


def _flush_acc_stripe(acc, out_hbm, cid, sid):
    for j in range(_ZJ):
        sl = pl.ds(sid * _STR + j * _C, _C)
        pltpu.sync_copy(acc.at[sl], out_hbm.at[cid, sl])
    sl = pl.ds(sid * _STR + _ZJ * _C, _ZT)
    pltpu.sync_copy(acc.at[sl], out_hbm.at[cid, sl])


def _relu_add(dst, src):
    def erow(r, carry):
        for k in range(_D // _L):
            sl = pl.ds(k * _L, _L)
            dst[r, sl] = jnp.maximum(dst[r, sl] + src[r, sl], 0.0)
        return carry

    lax.fori_loop(0, _C, erow, 0)


def _sc_segsum_body(data_hbm, idx_hbm, out_hbm, idxv,
                    r0, r1, r2, r3, l0, l1, l2, l3, s0, s1, s2, s3, acc):
    cid = lax.axis_index("c")
    sid = lax.axis_index("s")
    wid = cid * _NS + sid
    base = wid * _EW
    rows = (r0, r1, r2, r3)
    lsem = (l0, l1, l2, l3)
    ssem = (s0, s1, s2, s3)
    pltpu.sync_copy(idx_hbm.at[wid], idxv)
    _zero_acc_stripe(acc, r0, sid)
    for b in range(2):
        pltpu.async_copy(data_hbm.at[pl.ds(base + b * _C, _C)],
                         rows[b], lsem[b])
    plsc.subcore_barrier()

    def outer(c2, carry):
        for b in range(4):
            c = c2 * 4 + b
            pltpu.make_async_copy(data_hbm.at[pl.ds(base + c * _C, _C)],
                                  rows[b], lsem[b]).wait()
            pltpu.async_copy(rows[b], acc.at[idxv.at[c]], ssem[b], add=True)
            b2 = (b + 2) % 4

            def _wait_prev(b2=b2):
                pltpu.make_async_copy(
                    rows[b2], acc.at[idxv.at[c - 2]], ssem[b2]).wait()

            def _issue(b2=b2):
                pltpu.async_copy(
                    data_hbm.at[pl.ds(base + (c + 2) * _C, _C)],
                    rows[b2], lsem[b2])

            if b < 2:
                pl.when(c2 > 0)(_wait_prev)
                _issue()
            else:
                _wait_prev()
                pl.when(c2 < _K // 4 - 1)(_issue)
        return carry

    lax.fori_loop(0, _K // 4, outer, 0)
    for b in (2, 3):
        pltpu.make_async_copy(
            rows[b], acc.at[idxv.at[_K - 4 + b]], ssem[b]).wait()
    plsc.subcore_barrier()
    _flush_acc_stripe(acc, out_hbm, cid, sid)


def _sc_segsum(data, idx3):
    return pl.kernel(
        _sc_segsum_body,
        out_type=jax.ShapeDtypeStruct((_NC, _NV, _D), jnp.float32),
        mesh=_mesh(),
        compiler_params=_params(),
        scratch_types=[
            pltpu.VMEM((_K, _C), jnp.int32),
            pltpu.VMEM((_C, _D), jnp.float32),
            pltpu.VMEM((_C, _D), jnp.float32),
            pltpu.VMEM((_C, _D), jnp.float32),
            pltpu.VMEM((_C, _D), jnp.float32),
            pltpu.SemaphoreType.DMA,
            pltpu.SemaphoreType.DMA,
            pltpu.SemaphoreType.DMA,
            pltpu.SemaphoreType.DMA,
            pltpu.SemaphoreType.DMA,
            pltpu.SemaphoreType.DMA,
            pltpu.SemaphoreType.DMA,
            pltpu.SemaphoreType.DMA,
            pltpu.VMEM_SHARED((_NV, _D), jnp.float32),
        ],
    )(data, idx3)


def _sc_grs_body(a_hbm, b_hbm, gidx_hbm, sidx_hbm, m_hbm, out_hbm,
                 gidxv, sidxv, g0, g1, g2, g3, bb0, bb1,
                 gs0, gs1, gs2, gs3, ms0, ms1, ms2, ms3,
                 ss0, ss1, ss2, ss3, bs0, bs1, acc):
    cid = lax.axis_index("c")
    sid = lax.axis_index("s")
    wid = cid * _NS + sid
    base = wid * _EW
    g = (g0, g1, g2, g3)
    bb = (bb0, bb1)
    gsem = (gs0, gs1, gs2, gs3)
    msem = (ms0, ms1, ms2, ms3)
    ssem = (ss0, ss1, ss2, ss3)
    bsem = (bs0, bs1)
    pltpu.sync_copy(gidx_hbm.at[wid], gidxv)
    pltpu.sync_copy(sidx_hbm.at[wid], sidxv)
    _zero_acc_stripe(acc, g0, sid)
    for b in range(2):
        pltpu.async_copy(a_hbm.at[gidxv.at[b]], g[b], gsem[b])
        pltpu.async_copy(b_hbm.at[pl.ds(base + b * _C, _C)], bb[b], bsem[b])
    plsc.subcore_barrier()

    def outer(c2, carry):
        for b in range(4):
            bi = b % 2
            c = c2 * 4 + b
            pltpu.make_async_copy(a_hbm.at[gidxv.at[c]], g[b],
                                  gsem[b]).wait()
            pltpu.make_async_copy(b_hbm.at[pl.ds(base + c * _C, _C)],
                                  bb[bi], bsem[bi]).wait()
            _relu_add(g[b], bb[bi])

            def _issue_b(bi=bi):
                pltpu.async_copy(b_hbm.at[pl.ds(base + (c + 2) * _C, _C)],
                                 bb[bi], bsem[bi])

            if b < 2:
                _issue_b()
            else:
                pl.when(c2 < _K // 4 - 1)(_issue_b)
            pltpu.async_copy(g[b], m_hbm.at[pl.ds(base + c * _C, _C)],
                             msem[b])
            pltpu.async_copy(g[b], acc.at[sidxv.at[c]], ssem[b], add=True)
            b2 = (b + 2) % 4

            def _wait_prev(b2=b2):
                pltpu.make_async_copy(
                    g[b2], m_hbm.at[pl.ds(base + (c - 2) * _C, _C)],
                    msem[b2]).wait()
                pltpu.make_async_copy(
                    g[b2], acc.at[sidxv.at[c - 2]], ssem[b2]).wait()

            def _issue_g(b2=b2):
                pltpu.async_copy(a_hbm.at[gidxv.at[c + 2]], g[b2], gsem[b2])

            if b < 2:
                pl.when(c2 > 0)(_wait_prev)
                _issue_g()
            else:
                _wait_prev()
                pl.when(c2 < _K // 4 - 1)(_issue_g)
        return carry

    lax.fori_loop(0, _K // 4, outer, 0)
    for b in (2, 3):
        c = _K - 4 + b
        pltpu.make_async_copy(
            g[b], m_hbm.at[pl.ds(base + c * _C, _C)], msem[b]).wait()
        pltpu.make_async_copy(g[b], acc.at[sidxv.at[c]], ssem[b]).wait()
    plsc.subcore_barrier()
    _flush_acc_stripe(acc, out_hbm, cid, sid)


def _sc_gather_relu_scatter(a, b, gidx3, sidx3):
    return pl.kernel(
        _sc_grs_body,
        out_type=(jax.ShapeDtypeStruct((_E, _D), jnp.float32),
                  jax.ShapeDtypeStruct((_NC, _NF, _D), jnp.float32)),
        mesh=_mesh(),
        compiler_params=_params(),
        scratch_types=(
            [pltpu.VMEM((_K, _C), jnp.int32)] * 2
            + [pltpu.VMEM((_C, _D), jnp.float32)] * 6
            + [pltpu.SemaphoreType.DMA] * 14
            + [pltpu.VMEM_SHARED((_NF, _D), jnp.float32)]
        ),
    )(a, b, gidx3, sidx3)


def _sc_gr_body(a_hbm, b_hbm, gidx_hbm, m_hbm,
                gidxv, g0, g1, g2, g3, bb0, bb1,
                gs0, gs1, gs2, gs3, ms0, ms1, ms2, ms3,
                bs0, bs1, av_sh):
    cid = lax.axis_index("c")
    sid = lax.axis_index("s")
    wid = cid * _NS + sid
    base = wid * _EW
    g = (g0, g1, g2, g3)
    bb = (bb0, bb1)
    gsem = (gs0, gs1, gs2, gs3)
    msem = (ms0, ms1, ms2, ms3)
    bsem = (bs0, bs1)
    pltpu.sync_copy(gidx_hbm.at[wid], gidxv)
    for j in range(_ZJ):
        sl = pl.ds(sid * _STR + j * _C, _C)
        pltpu.sync_copy(a_hbm.at[sl], av_sh.at[sl])
    sl = pl.ds(sid * _STR + _ZJ * _C, _ZT)
    pltpu.sync_copy(a_hbm.at[sl], av_sh.at[sl])
    for b in range(2):
        pltpu.async_copy(b_hbm.at[pl.ds(base + b * _C, _C)], bb[b], bsem[b])
    plsc.subcore_barrier()
    for b in range(2):
        pltpu.async_copy(av_sh.at[gidxv.at[b]], g[b], gsem[b])

    def outer(c2, carry):
        for b in range(4):
            bi = b % 2
            c = c2 * 4 + b
            pltpu.make_async_copy(av_sh.at[gidxv.at[c]], g[b],
                                  gsem[b]).wait()
            pltpu.make_async_copy(b_hbm.at[pl.ds(base + c * _C, _C)],
                                  bb[bi], bsem[bi]).wait()
            _relu_add(g[b], bb[bi])

            def _issue_b(bi=bi):
                pltpu.async_copy(b_hbm.at[pl.ds(base + (c + 2) * _C, _C)],
                                 bb[bi], bsem[bi])

            if b < 2:
                _issue_b()
            else:
                pl.when(c2 < _K // 4 - 1)(_issue_b)
            pltpu.async_copy(g[b], m_hbm.at[pl.ds(base + c * _C, _C)],
                             msem[b])
            b2 = (b + 2) % 4

            def _wait_prev(b2=b2):
                pltpu.make_async_copy(
                    g[b2], m_hbm.at[pl.ds(base + (c - 2) * _C, _C)],
                    msem[b2]).wait()

            def _issue_g(b2=b2):
                pltpu.async_copy(av_sh.at[gidxv.at[c + 2]], g[b2], gsem[b2])

            if b < 2:
                pl.when(c2 > 0)(_wait_prev)
                _issue_g()
            else:
                _wait_prev()
                pl.when(c2 < _K // 4 - 1)(_issue_g)
        return carry

    lax.fori_loop(0, _K // 4, outer, 0)
    for b in (2, 3):
        c = _K - 4 + b
        pltpu.make_async_copy(
            g[b], m_hbm.at[pl.ds(base + c * _C, _C)], msem[b]).wait()


def _sc_gather_relu(a, b, gidx3):
    return pl.kernel(
        _sc_gr_body,
        out_type=jax.ShapeDtypeStruct((_E, _D), jnp.float32),
        mesh=_mesh(),
        compiler_params=_params(),
        scratch_types=(
            [pltpu.VMEM((_K, _C), jnp.int32)]
            + [pltpu.VMEM((_C, _D), jnp.float32)] * 6
            + [pltpu.SemaphoreType.DMA] * 10
            + [pltpu.VMEM_SHARED((_NF, _D), jnp.float32)]
        ),
    )(a, b, gidx3)


_EB = 3200


def _tc_cat_body(x_ref, y_ref, w_ref, o_ref):
    h = jnp.concatenate((x_ref[...], y_ref[...]), axis=-1)
    o_ref[...] = jnp.dot(h, w_ref[...], preferred_element_type=jnp.float32)


def _tc_edge_cat(x, y, wcat, eb=_EB):
    return pl.pallas_call(
        _tc_cat_body,
        grid=(_E // eb,),
        in_specs=[
            pl.BlockSpec((eb, _D), lambda i: (i, 0)),
            pl.BlockSpec((eb, _D), lambda i: (i, 0)),
            pl.BlockSpec((2 * _D, _D), lambda i: (0, 0)),
        ],
        out_specs=pl.BlockSpec((eb, _D), lambda i: (i, 0)),
        out_shape=jax.ShapeDtypeStruct((_E, _D), jnp.float32),
    )(x, y, wcat)


_NB = 2000


def _tc_node_body(s0_ref, s1_ref, x_ref, w1_ref, w2_ref, b_ref,
                  a_ref, nx_ref):
    s = s0_ref[0] + s1_ref[0]
    a_ref[...] = (
        jnp.dot(x_ref[...], w1_ref[...], preferred_element_type=jnp.float32)
        + jnp.dot(s, w2_ref[...], preferred_element_type=jnp.float32)
        + b_ref[...])
    nx_ref[...] = s + x_ref[...]


def _tc_node_combo(sp, x, w1, w2, b):
    n = x.shape[0]
    return pl.pallas_call(
        _tc_node_body,
        grid=(n // _NB,),
        in_specs=[
            pl.BlockSpec((1, _NB, _D), lambda i: (0, i, 0)),
            pl.BlockSpec((1, _NB, _D), lambda i: (1, i, 0)),
            pl.BlockSpec((_NB, _D), lambda i: (i, 0)),
            pl.BlockSpec((_D, _D), lambda i: (0, 0)),
            pl.BlockSpec((_D, _D), lambda i: (0, 0)),
            pl.BlockSpec((1, _D), lambda i: (0, 0)),
        ],
        out_specs=[
            pl.BlockSpec((_NB, _D), lambda i: (i, 0)),
            pl.BlockSpec((_NB, _D), lambda i: (i, 0)),
        ],
        out_shape=[
            jax.ShapeDtypeStruct((n, _D), jnp.float32),
            jax.ShapeDtypeStruct((n, _D), jnp.float32),
        ],
    )(sp, sp, x, w1, w2, b)


def kernel(variable, factor, edge_attr, prev_m_f_to_v, v_to_f, f_to_v,
           W_v, b_v, W_f, b_f):
    gidx = v_to_f.reshape(_NW, _K, _C)
    sidx = f_to_v.reshape(_NW, _K, _C)
    Wv1, Wv2, Wv3 = W_v[:_D], W_v[_D:2 * _D], W_v[2 * _D:]
    Wf1, Wf2, Wf3 = W_f[:_D], W_f[_D:2 * _D], W_f[2 * _D:]

    s1p = _sc_segsum(prev_m_f_to_v, gidx)
    b1 = _tc_edge_cat(edge_attr, prev_m_f_to_v,
                      jnp.concatenate((Wv3, -Wv2), axis=0))
    a_v, new_variable = _tc_node_combo(s1p, variable, Wv1, Wv2,
                                       b_v.reshape(1, _D))
    m_v_to_f, s2p = _sc_gather_relu_scatter(a_v, b1, gidx, sidx)
    a_f, new_factor = _tc_node_combo(s2p, factor, Wf1, Wf2,
                                     b_f.reshape(1, _D))
    p2 = _tc_edge_cat(edge_attr, m_v_to_f,
                      jnp.concatenate((Wf3, -Wf2), axis=0))
    m_f_to_v = _sc_gather_relu(a_f, p2, sidx)
    return (m_f_to_v, new_factor, new_variable)


# speedup vs baseline: 2.1189x; 1.0159x over previous
import jax
import jax.numpy as jnp
from jax import lax
from jax.experimental import pallas as pl
from jax.experimental.pallas import tpu as pltpu
from jax.experimental.pallas import tpu_sc as plsc

_NV = 10000
_NF = 10000
_E = 160000
_D = 128

_NC = 2
_NS = 16
_NW = _NC * _NS
_L = 16

_EW = _E // _NW
_C = 50
_K = _EW // _C
_STR = _NV // _NS
_ZJ = _STR // _C
_ZT = _STR - _ZJ * _C


def _mesh():
    return plsc.VectorSubcoreMesh(
        core_axis_name="c", subcore_axis_name="s",
        num_cores=_NC, num_subcores=_NS)


def _params():
    return pltpu.CompilerParams(use_tc_tiling_on_sc=False)


def _zero_buf(buf):
    zv = jnp.zeros((_L,), jnp.float32)

    def zrow(r, carry):
        for k in range(_D // _L):
            buf[r, pl.ds(k * _L, _L)] = zv
        return carry

    lax.fori_loop(0, _C, zrow, 0)


def _zero_acc_stripe(acc, zbuf, sid):
    _zero_buf(zbuf)
    for j in range(_ZJ):
        pltpu.sync_copy(zbuf, acc.at[pl.ds(sid * _STR + j * _C, _C)])
    pltpu.sync_copy(zbuf.at[pl.ds(0, _ZT)],
                    acc.at[pl.ds(sid * _STR + _ZJ * _C, _ZT)])


def _flush_acc_stripe(acc, out_hbm, cid, sid):
    for j in range(_ZJ):
        sl = pl.ds(sid * _STR + j * _C, _C)
        pltpu.sync_copy(acc.at[sl], out_hbm.at[cid, sl])
    sl = pl.ds(sid * _STR + _ZJ * _C, _ZT)
    pltpu.sync_copy(acc.at[sl], out_hbm.at[cid, sl])


def _relu_add(dst, src):
    def erow(r, carry):
        for k in range(_D // _L):
            sl = pl.ds(k * _L, _L)
            dst[r, sl] = jnp.maximum(dst[r, sl] + src[r, sl], 0.0)
        return carry

    lax.fori_loop(0, _C, erow, 0)


def _sc_segsum_body(data_hbm, idx_hbm, out_hbm, idxv,
                    r0, r1, r2, r3, l0, l1, l2, l3, s0, s1, s2, s3, acc):
    cid = lax.axis_index("c")
    sid = lax.axis_index("s")
    wid = cid * _NS + sid
    base = wid * _EW
    rows = (r0, r1, r2, r3)
    lsem = (l0, l1, l2, l3)
    ssem = (s0, s1, s2, s3)
    pltpu.sync_copy(idx_hbm.at[wid], idxv)
    _zero_acc_stripe(acc, r0, sid)
    for b in range(2):
        pltpu.async_copy(data_hbm.at[pl.ds(base + b * _C, _C)],
                         rows[b], lsem[b])
    plsc.subcore_barrier()

    def outer(c2, carry):
        for b in range(4):
            c = c2 * 4 + b
            pltpu.make_async_copy(data_hbm.at[pl.ds(base + c * _C, _C)],
                                  rows[b], lsem[b]).wait()
            pltpu.async_copy(rows[b], acc.at[idxv.at[c]], ssem[b], add=True)
            b2 = (b + 2) % 4

            def _wait_prev(b2=b2):
                pltpu.make_async_copy(
                    rows[b2], acc.at[idxv.at[c - 2]], ssem[b2]).wait()

            def _issue(b2=b2):
                pltpu.async_copy(
                    data_hbm.at[pl.ds(base + (c + 2) * _C, _C)],
                    rows[b2], lsem[b2])

            if b < 2:
                pl.when(c2 > 0)(_wait_prev)
                _issue()
            else:
                _wait_prev()
                pl.when(c2 < _K // 4 - 1)(_issue)
        return carry

    lax.fori_loop(0, _K // 4, outer, 0)
    for b in (2, 3):
        pltpu.make_async_copy(
            rows[b], acc.at[idxv.at[_K - 4 + b]], ssem[b]).wait()
    plsc.subcore_barrier()
    _flush_acc_stripe(acc, out_hbm, cid, sid)


def _sc_segsum(data, idx3):
    return pl.kernel(
        _sc_segsum_body,
        out_type=jax.ShapeDtypeStruct((_NC, _NV, _D), jnp.float32),
        mesh=_mesh(),
        compiler_params=_params(),
        scratch_types=[
            pltpu.VMEM((_K, _C), jnp.int32),
            pltpu.VMEM((_C, _D), jnp.float32),
            pltpu.VMEM((_C, _D), jnp.float32),
            pltpu.VMEM((_C, _D), jnp.float32),
            pltpu.VMEM((_C, _D), jnp.float32),
            pltpu.SemaphoreType.DMA,
            pltpu.SemaphoreType.DMA,
            pltpu.SemaphoreType.DMA,
            pltpu.SemaphoreType.DMA,
            pltpu.SemaphoreType.DMA,
            pltpu.SemaphoreType.DMA,
            pltpu.SemaphoreType.DMA,
            pltpu.SemaphoreType.DMA,
            pltpu.VMEM_SHARED((_NV, _D), jnp.float32),
        ],
    )(data, idx3)


def _sc_grs_body(a_hbm, b_hbm, gidx_hbm, sidx_hbm, m_hbm, out_hbm,
                 gidxv, sidxv, g0, g1, g2, g3, bb0, bb1,
                 gs0, gs1, gs2, gs3, ms0, ms1, ms2, ms3,
                 ss0, ss1, ss2, ss3, bs0, bs1, acc):
    cid = lax.axis_index("c")
    sid = lax.axis_index("s")
    wid = cid * _NS + sid
    base = wid * _EW
    g = (g0, g1, g2, g3)
    bb = (bb0, bb1)
    gsem = (gs0, gs1, gs2, gs3)
    msem = (ms0, ms1, ms2, ms3)
    ssem = (ss0, ss1, ss2, ss3)
    bsem = (bs0, bs1)
    pltpu.sync_copy(gidx_hbm.at[wid], gidxv)
    pltpu.sync_copy(sidx_hbm.at[wid], sidxv)
    _zero_acc_stripe(acc, g0, sid)
    for b in range(2):
        pltpu.async_copy(a_hbm.at[gidxv.at[b]], g[b], gsem[b])
        pltpu.async_copy(b_hbm.at[pl.ds(base + b * _C, _C)], bb[b], bsem[b])
    plsc.subcore_barrier()

    def outer(c2, carry):
        for b in range(4):
            bi = b % 2
            c = c2 * 4 + b
            pltpu.make_async_copy(a_hbm.at[gidxv.at[c]], g[b],
                                  gsem[b]).wait()
            pltpu.make_async_copy(b_hbm.at[pl.ds(base + c * _C, _C)],
                                  bb[bi], bsem[bi]).wait()
            _relu_add(g[b], bb[bi])

            def _issue_b(bi=bi):
                pltpu.async_copy(b_hbm.at[pl.ds(base + (c + 2) * _C, _C)],
                                 bb[bi], bsem[bi])

            if b < 2:
                _issue_b()
            else:
                pl.when(c2 < _K // 4 - 1)(_issue_b)
            pltpu.async_copy(g[b], m_hbm.at[pl.ds(base + c * _C, _C)],
                             msem[b])
            pltpu.async_copy(g[b], acc.at[sidxv.at[c]], ssem[b], add=True)
            b2 = (b + 2) % 4

            def _wait_prev(b2=b2):
                pltpu.make_async_copy(
                    g[b2], m_hbm.at[pl.ds(base + (c - 2) * _C, _C)],
                    msem[b2]).wait()
                pltpu.make_async_copy(
                    g[b2], acc.at[sidxv.at[c - 2]], ssem[b2]).wait()

            def _issue_g(b2=b2):
                pltpu.async_copy(a_hbm.at[gidxv.at[c + 2]], g[b2], gsem[b2])

            if b < 2:
                pl.when(c2 > 0)(_wait_prev)
                _issue_g()
            else:
                _wait_prev()
                pl.when(c2 < _K // 4 - 1)(_issue_g)
        return carry

    lax.fori_loop(0, _K // 4, outer, 0)
    for b in (2, 3):
        c = _K - 4 + b
        pltpu.make_async_copy(
            g[b], m_hbm.at[pl.ds(base + c * _C, _C)], msem[b]).wait()
        pltpu.make_async_copy(g[b], acc.at[sidxv.at[c]], ssem[b]).wait()
    plsc.subcore_barrier()
    _flush_acc_stripe(acc, out_hbm, cid, sid)


def _sc_gather_relu_scatter(a, b, gidx3, sidx3):
    return pl.kernel(
        _sc_grs_body,
        out_type=(jax.ShapeDtypeStruct((_E, _D), jnp.float32),
                  jax.ShapeDtypeStruct((_NC, _NF, _D), jnp.float32)),
        mesh=_mesh(),
        compiler_params=_params(),
        scratch_types=(
            [pltpu.VMEM((_K, _C), jnp.int32)] * 2
            + [pltpu.VMEM((_C, _D), jnp.float32)] * 6
            + [pltpu.SemaphoreType.DMA] * 14
            + [pltpu.VMEM_SHARED((_NF, _D), jnp.float32)]
        ),
    )(a, b, gidx3, sidx3)


def _sc_gr_body(a_hbm, b_hbm, gidx_hbm, m_hbm,
                gidxv, g0, g1, g2, g3, bb0, bb1,
                gs0, gs1, gs2, gs3, ms0, ms1, ms2, ms3,
                bs0, bs1, av_sh):
    cid = lax.axis_index("c")
    sid = lax.axis_index("s")
    wid = cid * _NS + sid
    base = wid * _EW
    g = (g0, g1, g2, g3)
    bb = (bb0, bb1)
    gsem = (gs0, gs1, gs2, gs3)
    msem = (ms0, ms1, ms2, ms3)
    bsem = (bs0, bs1)
    pltpu.sync_copy(gidx_hbm.at[wid], gidxv)
    for j in range(_ZJ):
        sl = pl.ds(sid * _STR + j * _C, _C)
        pltpu.sync_copy(a_hbm.at[sl], av_sh.at[sl])
    sl = pl.ds(sid * _STR + _ZJ * _C, _ZT)
    pltpu.sync_copy(a_hbm.at[sl], av_sh.at[sl])
    for b in range(2):
        pltpu.async_copy(b_hbm.at[pl.ds(base + b * _C, _C)], bb[b], bsem[b])
    plsc.subcore_barrier()
    for b in range(2):
        pltpu.async_copy(av_sh.at[gidxv.at[b]], g[b], gsem[b])

    def outer(c2, carry):
        for b in range(4):
            bi = b % 2
            c = c2 * 4 + b
            pltpu.make_async_copy(av_sh.at[gidxv.at[c]], g[b],
                                  gsem[b]).wait()
            pltpu.make_async_copy(b_hbm.at[pl.ds(base + c * _C, _C)],
                                  bb[bi], bsem[bi]).wait()
            _relu_add(g[b], bb[bi])

            def _issue_b(bi=bi):
                pltpu.async_copy(b_hbm.at[pl.ds(base + (c + 2) * _C, _C)],
                                 bb[bi], bsem[bi])

            if b < 2:
                _issue_b()
            else:
                pl.when(c2 < _K // 4 - 1)(_issue_b)
            pltpu.async_copy(g[b], m_hbm.at[pl.ds(base + c * _C, _C)],
                             msem[b])
            b2 = (b + 2) % 4

            def _wait_prev(b2=b2):
                pltpu.make_async_copy(
                    g[b2], m_hbm.at[pl.ds(base + (c - 2) * _C, _C)],
                    msem[b2]).wait()

            def _issue_g(b2=b2):
                pltpu.async_copy(av_sh.at[gidxv.at[c + 2]], g[b2], gsem[b2])

            if b < 2:
                pl.when(c2 > 0)(_wait_prev)
                _issue_g()
            else:
                _wait_prev()
                pl.when(c2 < _K // 4 - 1)(_issue_g)
        return carry

    lax.fori_loop(0, _K // 4, outer, 0)
    for b in (2, 3):
        c = _K - 4 + b
        pltpu.make_async_copy(
            g[b], m_hbm.at[pl.ds(base + c * _C, _C)], msem[b]).wait()


def _sc_gather_relu(a, b, gidx3):
    return pl.kernel(
        _sc_gr_body,
        out_type=jax.ShapeDtypeStruct((_E, _D), jnp.float32),
        mesh=_mesh(),
        compiler_params=_params(),
        scratch_types=(
            [pltpu.VMEM((_K, _C), jnp.int32)]
            + [pltpu.VMEM((_C, _D), jnp.float32)] * 6
            + [pltpu.SemaphoreType.DMA] * 10
            + [pltpu.VMEM_SHARED((_NF, _D), jnp.float32)]
        ),
    )(a, b, gidx3)


_EB = 8000


def _tc_cat_body(x_ref, y_ref, w_ref, o_ref):
    h = jnp.concatenate((x_ref[...], y_ref[...]), axis=-1)
    o_ref[...] = jnp.dot(h, w_ref[...], preferred_element_type=jnp.float32)


def _tc_edge_cat(x, y, wcat, eb=_EB):
    return pl.pallas_call(
        _tc_cat_body,
        grid=(_E // eb,),
        in_specs=[
            pl.BlockSpec((eb, _D), lambda i: (i, 0)),
            pl.BlockSpec((eb, _D), lambda i: (i, 0)),
            pl.BlockSpec((2 * _D, _D), lambda i: (0, 0)),
        ],
        out_specs=pl.BlockSpec((eb, _D), lambda i: (i, 0)),
        out_shape=jax.ShapeDtypeStruct((_E, _D), jnp.float32),
    )(x, y, wcat)


_NB = 2000


def _tc_node_body(s0_ref, s1_ref, x_ref, w1_ref, w2_ref, b_ref,
                  a_ref, nx_ref):
    s = s0_ref[0] + s1_ref[0]
    a_ref[...] = (
        jnp.dot(x_ref[...], w1_ref[...], preferred_element_type=jnp.float32)
        + jnp.dot(s, w2_ref[...], preferred_element_type=jnp.float32)
        + b_ref[...])
    nx_ref[...] = s + x_ref[...]


def _tc_node_combo(sp, x, w1, w2, b):
    n = x.shape[0]
    return pl.pallas_call(
        _tc_node_body,
        grid=(n // _NB,),
        in_specs=[
            pl.BlockSpec((1, _NB, _D), lambda i: (0, i, 0)),
            pl.BlockSpec((1, _NB, _D), lambda i: (1, i, 0)),
            pl.BlockSpec((_NB, _D), lambda i: (i, 0)),
            pl.BlockSpec((_D, _D), lambda i: (0, 0)),
            pl.BlockSpec((_D, _D), lambda i: (0, 0)),
            pl.BlockSpec((1, _D), lambda i: (0, 0)),
        ],
        out_specs=[
            pl.BlockSpec((_NB, _D), lambda i: (i, 0)),
            pl.BlockSpec((_NB, _D), lambda i: (i, 0)),
        ],
        out_shape=[
            jax.ShapeDtypeStruct((n, _D), jnp.float32),
            jax.ShapeDtypeStruct((n, _D), jnp.float32),
        ],
    )(sp, sp, x, w1, w2, b)


def kernel(variable, factor, edge_attr, prev_m_f_to_v, v_to_f, f_to_v,
           W_v, b_v, W_f, b_f):
    gidx = v_to_f.reshape(_NW, _K, _C)
    sidx = f_to_v.reshape(_NW, _K, _C)
    Wv1, Wv2, Wv3 = W_v[:_D], W_v[_D:2 * _D], W_v[2 * _D:]
    Wf1, Wf2, Wf3 = W_f[:_D], W_f[_D:2 * _D], W_f[2 * _D:]

    s1p = _sc_segsum(prev_m_f_to_v, gidx)
    b1 = _tc_edge_cat(edge_attr, prev_m_f_to_v,
                      jnp.concatenate((Wv3, -Wv2), axis=0))
    a_v, new_variable = _tc_node_combo(s1p, variable, Wv1, Wv2,
                                       b_v.reshape(1, _D))
    m_v_to_f, s2p = _sc_gather_relu_scatter(a_v, b1, gidx, sidx)
    a_f, new_factor = _tc_node_combo(s2p, factor, Wf1, Wf2,
                                     b_f.reshape(1, _D))
    p2 = _tc_edge_cat(edge_attr, m_v_to_f,
                      jnp.concatenate((Wf3, -Wf2), axis=0))
    m_f_to_v = _sc_gather_relu(a_f, p2, sidx)
    return (m_f_to_v, new_factor, new_variable)


# speedup vs baseline: 2.1273x; 1.0039x over previous
import jax
import jax.numpy as jnp
from jax import lax
from jax.experimental import pallas as pl
from jax.experimental.pallas import tpu as pltpu
from jax.experimental.pallas import tpu_sc as plsc

_NV = 10000
_NF = 10000
_E = 160000
_D = 128

_NC = 2
_NS = 16
_NW = _NC * _NS
_L = 16

_EW = _E // _NW
_C = 50
_K = _EW // _C
_STR = _NV // _NS
_ZJ = _STR // _C
_ZT = _STR - _ZJ * _C


def _mesh():
    return plsc.VectorSubcoreMesh(
        core_axis_name="c", subcore_axis_name="s",
        num_cores=_NC, num_subcores=_NS)


def _params():
    return pltpu.CompilerParams(use_tc_tiling_on_sc=False)


def _zero_buf(buf):
    zv = jnp.zeros((_L,), jnp.float32)

    def zrow(r, carry):
        for k in range(_D // _L):
            buf[r, pl.ds(k * _L, _L)] = zv
        return carry

    lax.fori_loop(0, _C, zrow, 0)


def _zero_acc_stripe(acc, zbuf, sid):
    _zero_buf(zbuf)
    for j in range(_ZJ):
        pltpu.sync_copy(zbuf, acc.at[pl.ds(sid * _STR + j * _C, _C)])
    pltpu.sync_copy(zbuf.at[pl.ds(0, _ZT)],
                    acc.at[pl.ds(sid * _STR + _ZJ * _C, _ZT)])


def _flush_acc_stripe(acc, out_hbm, cid, sid):
    for j in range(_ZJ):
        sl = pl.ds(sid * _STR + j * _C, _C)
        pltpu.sync_copy(acc.at[sl], out_hbm.at[cid, sl])
    sl = pl.ds(sid * _STR + _ZJ * _C, _ZT)
    pltpu.sync_copy(acc.at[sl], out_hbm.at[cid, sl])


def _relu_add(dst, src):
    def erow(r, carry):
        for k in range(_D // _L):
            sl = pl.ds(k * _L, _L)
            dst[r, sl] = jnp.maximum(dst[r, sl] + src[r, sl], 0.0)
        return carry

    lax.fori_loop(0, _C, erow, 0)


def _sc_segsum_body(data_hbm, idx_hbm, out_hbm, idxv,
                    r0, r1, r2, r3, l0, l1, l2, l3, s0, s1, s2, s3, acc):
    cid = lax.axis_index("c")
    sid = lax.axis_index("s")
    wid = cid * _NS + sid
    base = wid * _EW
    rows = (r0, r1, r2, r3)
    lsem = (l0, l1, l2, l3)
    ssem = (s0, s1, s2, s3)
    pltpu.sync_copy(idx_hbm.at[wid], idxv)
    _zero_acc_stripe(acc, r0, sid)
    for b in range(2):
        pltpu.async_copy(data_hbm.at[pl.ds(base + b * _C, _C)],
                         rows[b], lsem[b])
    plsc.subcore_barrier()

    def outer(c2, carry):
        for b in range(4):
            c = c2 * 4 + b
            pltpu.make_async_copy(data_hbm.at[pl.ds(base + c * _C, _C)],
                                  rows[b], lsem[b]).wait()
            pltpu.async_copy(rows[b], acc.at[idxv.at[c]], ssem[b], add=True)
            b2 = (b + 2) % 4

            def _wait_prev(b2=b2):
                pltpu.make_async_copy(
                    rows[b2], acc.at[idxv.at[c - 2]], ssem[b2]).wait()

            def _issue(b2=b2):
                pltpu.async_copy(
                    data_hbm.at[pl.ds(base + (c + 2) * _C, _C)],
                    rows[b2], lsem[b2])

            if b < 2:
                pl.when(c2 > 0)(_wait_prev)
                _issue()
            else:
                _wait_prev()
                pl.when(c2 < _K // 4 - 1)(_issue)
        return carry

    lax.fori_loop(0, _K // 4, outer, 0)
    for b in (2, 3):
        pltpu.make_async_copy(
            rows[b], acc.at[idxv.at[_K - 4 + b]], ssem[b]).wait()
    plsc.subcore_barrier()
    _flush_acc_stripe(acc, out_hbm, cid, sid)


def _sc_segsum(data, idx3):
    return pl.kernel(
        _sc_segsum_body,
        out_type=jax.ShapeDtypeStruct((_NC, _NV, _D), jnp.float32),
        mesh=_mesh(),
        compiler_params=_params(),
        scratch_types=[
            pltpu.VMEM((_K, _C), jnp.int32),
            pltpu.VMEM((_C, _D), jnp.float32),
            pltpu.VMEM((_C, _D), jnp.float32),
            pltpu.VMEM((_C, _D), jnp.float32),
            pltpu.VMEM((_C, _D), jnp.float32),
            pltpu.SemaphoreType.DMA,
            pltpu.SemaphoreType.DMA,
            pltpu.SemaphoreType.DMA,
            pltpu.SemaphoreType.DMA,
            pltpu.SemaphoreType.DMA,
            pltpu.SemaphoreType.DMA,
            pltpu.SemaphoreType.DMA,
            pltpu.SemaphoreType.DMA,
            pltpu.VMEM_SHARED((_NV, _D), jnp.float32),
        ],
    )(data, idx3)


def _sc_grs_body(a_hbm, b_hbm, gidx_hbm, sidx_hbm, m_hbm, out_hbm,
                 gidxv, sidxv, g0, g1, g2, g3, bb0, bb1,
                 gs0, gs1, gs2, gs3, ms0, ms1, ms2, ms3,
                 ss0, ss1, ss2, ss3, bs0, bs1, acc):
    cid = lax.axis_index("c")
    sid = lax.axis_index("s")
    wid = cid * _NS + sid
    base = wid * _EW
    g = (g0, g1, g2, g3)
    bb = (bb0, bb1)
    gsem = (gs0, gs1, gs2, gs3)
    msem = (ms0, ms1, ms2, ms3)
    ssem = (ss0, ss1, ss2, ss3)
    bsem = (bs0, bs1)
    pltpu.sync_copy(gidx_hbm.at[wid], gidxv)
    pltpu.sync_copy(sidx_hbm.at[wid], sidxv)
    _zero_acc_stripe(acc, g0, sid)
    for b in range(2):
        pltpu.async_copy(a_hbm.at[gidxv.at[b]], g[b], gsem[b])
        pltpu.async_copy(b_hbm.at[pl.ds(base + b * _C, _C)], bb[b], bsem[b])
    plsc.subcore_barrier()

    def outer(c2, carry):
        for b in range(4):
            bi = b % 2
            c = c2 * 4 + b
            pltpu.make_async_copy(a_hbm.at[gidxv.at[c]], g[b],
                                  gsem[b]).wait()
            pltpu.make_async_copy(b_hbm.at[pl.ds(base + c * _C, _C)],
                                  bb[bi], bsem[bi]).wait()
            _relu_add(g[b], bb[bi])

            def _issue_b(bi=bi):
                pltpu.async_copy(b_hbm.at[pl.ds(base + (c + 2) * _C, _C)],
                                 bb[bi], bsem[bi])

            if b < 2:
                _issue_b()
            else:
                pl.when(c2 < _K // 4 - 1)(_issue_b)
            pltpu.async_copy(g[b], m_hbm.at[pl.ds(base + c * _C, _C)],
                             msem[b])
            pltpu.async_copy(g[b], acc.at[sidxv.at[c]], ssem[b], add=True)
            b2 = (b + 2) % 4

            def _wait_prev(b2=b2):
                pltpu.make_async_copy(
                    g[b2], m_hbm.at[pl.ds(base + (c - 2) * _C, _C)],
                    msem[b2]).wait()
                pltpu.make_async_copy(
                    g[b2], acc.at[sidxv.at[c - 2]], ssem[b2]).wait()

            def _issue_g(b2=b2):
                pltpu.async_copy(a_hbm.at[gidxv.at[c + 2]], g[b2], gsem[b2])

            if b < 2:
                pl.when(c2 > 0)(_wait_prev)
                _issue_g()
            else:
                _wait_prev()
                pl.when(c2 < _K // 4 - 1)(_issue_g)
        return carry

    lax.fori_loop(0, _K // 4, outer, 0)
    for b in (2, 3):
        c = _K - 4 + b
        pltpu.make_async_copy(
            g[b], m_hbm.at[pl.ds(base + c * _C, _C)], msem[b]).wait()
        pltpu.make_async_copy(g[b], acc.at[sidxv.at[c]], ssem[b]).wait()
    plsc.subcore_barrier()
    _flush_acc_stripe(acc, out_hbm, cid, sid)


def _sc_gather_relu_scatter(a, b, gidx3, sidx3):
    return pl.kernel(
        _sc_grs_body,
        out_type=(jax.ShapeDtypeStruct((_E, _D), jnp.float32),
                  jax.ShapeDtypeStruct((_NC, _NF, _D), jnp.float32)),
        mesh=_mesh(),
        compiler_params=_params(),
        scratch_types=(
            [pltpu.VMEM((_K, _C), jnp.int32)] * 2
            + [pltpu.VMEM((_C, _D), jnp.float32)] * 6
            + [pltpu.SemaphoreType.DMA] * 14
            + [pltpu.VMEM_SHARED((_NF, _D), jnp.float32)]
        ),
    )(a, b, gidx3, sidx3)


def _sc_gr_body(a_hbm, b_hbm, gidx_hbm, m_hbm,
                gidxv, g0, g1, g2, g3, bb0, bb1,
                gs0, gs1, gs2, gs3, ms0, ms1, ms2, ms3,
                bs0, bs1, av_sh):
    cid = lax.axis_index("c")
    sid = lax.axis_index("s")
    wid = cid * _NS + sid
    base = wid * _EW
    g = (g0, g1, g2, g3)
    bb = (bb0, bb1)
    gsem = (gs0, gs1, gs2, gs3)
    msem = (ms0, ms1, ms2, ms3)
    bsem = (bs0, bs1)
    pltpu.sync_copy(gidx_hbm.at[wid], gidxv)
    for j in range(_ZJ):
        sl = pl.ds(sid * _STR + j * _C, _C)
        pltpu.sync_copy(a_hbm.at[sl], av_sh.at[sl])
    sl = pl.ds(sid * _STR + _ZJ * _C, _ZT)
    pltpu.sync_copy(a_hbm.at[sl], av_sh.at[sl])
    for b in range(2):
        pltpu.async_copy(b_hbm.at[pl.ds(base + b * _C, _C)], bb[b], bsem[b])
    plsc.subcore_barrier()
    for b in range(2):
        pltpu.async_copy(av_sh.at[gidxv.at[b]], g[b], gsem[b])

    def outer(c2, carry):
        for b in range(4):
            bi = b % 2
            c = c2 * 4 + b
            pltpu.make_async_copy(av_sh.at[gidxv.at[c]], g[b],
                                  gsem[b]).wait()
            pltpu.make_async_copy(b_hbm.at[pl.ds(base + c * _C, _C)],
                                  bb[bi], bsem[bi]).wait()
            _relu_add(g[b], bb[bi])

            def _issue_b(bi=bi):
                pltpu.async_copy(b_hbm.at[pl.ds(base + (c + 2) * _C, _C)],
                                 bb[bi], bsem[bi])

            if b < 2:
                _issue_b()
            else:
                pl.when(c2 < _K // 4 - 1)(_issue_b)
            pltpu.async_copy(g[b], m_hbm.at[pl.ds(base + c * _C, _C)],
                             msem[b])
            b2 = (b + 2) % 4

            def _wait_prev(b2=b2):
                pltpu.make_async_copy(
                    g[b2], m_hbm.at[pl.ds(base + (c - 2) * _C, _C)],
                    msem[b2]).wait()

            def _issue_g(b2=b2):
                pltpu.async_copy(av_sh.at[gidxv.at[c + 2]], g[b2], gsem[b2])

            if b < 2:
                pl.when(c2 > 0)(_wait_prev)
                _issue_g()
            else:
                _wait_prev()
                pl.when(c2 < _K // 4 - 1)(_issue_g)
        return carry

    lax.fori_loop(0, _K // 4, outer, 0)
    for b in (2, 3):
        c = _K - 4 + b
        pltpu.make_async_copy(
            g[b], m_hbm.at[pl.ds(base + c * _C, _C)], msem[b]).wait()


def _sc_gather_relu(a, b, gidx3):
    return pl.kernel(
        _sc_gr_body,
        out_type=jax.ShapeDtypeStruct((_E, _D), jnp.float32),
        mesh=_mesh(),
        compiler_params=_params(),
        scratch_types=(
            [pltpu.VMEM((_K, _C), jnp.int32)]
            + [pltpu.VMEM((_C, _D), jnp.float32)] * 6
            + [pltpu.SemaphoreType.DMA] * 10
            + [pltpu.VMEM_SHARED((_NF, _D), jnp.float32)]
        ),
    )(a, b, gidx3)


_EB = 16000


def _tc_cat_body(x_ref, y_ref, w_ref, o_ref):
    h = jnp.concatenate((x_ref[...], y_ref[...]), axis=-1)
    o_ref[...] = jnp.dot(h, w_ref[...], preferred_element_type=jnp.float32)


def _tc_edge_cat(x, y, wcat, eb=_EB):
    return pl.pallas_call(
        _tc_cat_body,
        grid=(_E // eb,),
        in_specs=[
            pl.BlockSpec((eb, _D), lambda i: (i, 0)),
            pl.BlockSpec((eb, _D), lambda i: (i, 0)),
            pl.BlockSpec((2 * _D, _D), lambda i: (0, 0)),
        ],
        out_specs=pl.BlockSpec((eb, _D), lambda i: (i, 0)),
        out_shape=jax.ShapeDtypeStruct((_E, _D), jnp.float32),
    )(x, y, wcat)


_NB = 2000


def _tc_node_body(s0_ref, s1_ref, x_ref, w1_ref, w2_ref, b_ref,
                  a_ref, nx_ref):
    s = s0_ref[0] + s1_ref[0]
    a_ref[...] = (
        jnp.dot(x_ref[...], w1_ref[...], preferred_element_type=jnp.float32)
        + jnp.dot(s, w2_ref[...], preferred_element_type=jnp.float32)
        + b_ref[...])
    nx_ref[...] = s + x_ref[...]


def _tc_node_combo(sp, x, w1, w2, b):
    n = x.shape[0]
    return pl.pallas_call(
        _tc_node_body,
        grid=(n // _NB,),
        in_specs=[
            pl.BlockSpec((1, _NB, _D), lambda i: (0, i, 0)),
            pl.BlockSpec((1, _NB, _D), lambda i: (1, i, 0)),
            pl.BlockSpec((_NB, _D), lambda i: (i, 0)),
            pl.BlockSpec((_D, _D), lambda i: (0, 0)),
            pl.BlockSpec((_D, _D), lambda i: (0, 0)),
            pl.BlockSpec((1, _D), lambda i: (0, 0)),
        ],
        out_specs=[
            pl.BlockSpec((_NB, _D), lambda i: (i, 0)),
            pl.BlockSpec((_NB, _D), lambda i: (i, 0)),
        ],
        out_shape=[
            jax.ShapeDtypeStruct((n, _D), jnp.float32),
            jax.ShapeDtypeStruct((n, _D), jnp.float32),
        ],
    )(sp, sp, x, w1, w2, b)


def kernel(variable, factor, edge_attr, prev_m_f_to_v, v_to_f, f_to_v,
           W_v, b_v, W_f, b_f):
    gidx = v_to_f.reshape(_NW, _K, _C)
    sidx = f_to_v.reshape(_NW, _K, _C)
    Wv1, Wv2, Wv3 = W_v[:_D], W_v[_D:2 * _D], W_v[2 * _D:]
    Wf1, Wf2, Wf3 = W_f[:_D], W_f[_D:2 * _D], W_f[2 * _D:]

    s1p = _sc_segsum(prev_m_f_to_v, gidx)
    b1 = _tc_edge_cat(edge_attr, prev_m_f_to_v,
                      jnp.concatenate((Wv3, -Wv2), axis=0))
    a_v, new_variable = _tc_node_combo(s1p, variable, Wv1, Wv2,
                                       b_v.reshape(1, _D))
    m_v_to_f, s2p = _sc_gather_relu_scatter(a_v, b1, gidx, sidx)
    a_f, new_factor = _tc_node_combo(s2p, factor, Wf1, Wf2,
                                     b_f.reshape(1, _D))
    p2 = _tc_edge_cat(edge_attr, m_v_to_f,
                      jnp.concatenate((Wf3, -Wf2), axis=0))
    m_f_to_v = _sc_gather_relu(a_f, p2, sidx)
    return (m_f_to_v, new_factor, new_variable)


# speedup vs baseline: 2.1283x; 1.0005x over previous
import jax
import jax.numpy as jnp
from jax import lax
from jax.experimental import pallas as pl
from jax.experimental.pallas import tpu as pltpu
from jax.experimental.pallas import tpu_sc as plsc

_NV = 10000
_NF = 10000
_E = 160000
_D = 128

_NC = 2
_NS = 16
_NW = _NC * _NS
_L = 16

_EW = _E // _NW
_C = 50
_K = _EW // _C
_STR = _NV // _NS
_ZJ = _STR // _C
_ZT = _STR - _ZJ * _C


def _mesh():
    return plsc.VectorSubcoreMesh(
        core_axis_name="c", subcore_axis_name="s",
        num_cores=_NC, num_subcores=_NS)


def _params():
    return pltpu.CompilerParams(use_tc_tiling_on_sc=False)


def _zero_buf(buf):
    zv = jnp.zeros((_L,), jnp.float32)

    def zrow(r, carry):
        for k in range(_D // _L):
            buf[r, pl.ds(k * _L, _L)] = zv
        return carry

    lax.fori_loop(0, _C, zrow, 0)


def _zero_acc_stripe(acc, zbuf, sid):
    _zero_buf(zbuf)
    for j in range(_ZJ):
        pltpu.sync_copy(zbuf, acc.at[pl.ds(sid * _STR + j * _C, _C)])
    pltpu.sync_copy(zbuf.at[pl.ds(0, _ZT)],
                    acc.at[pl.ds(sid * _STR + _ZJ * _C, _ZT)])


def _flush_acc_stripe(acc, out_hbm, cid, sid):
    for j in range(_ZJ):
        sl = pl.ds(sid * _STR + j * _C, _C)
        pltpu.sync_copy(acc.at[sl], out_hbm.at[cid, sl])
    sl = pl.ds(sid * _STR + _ZJ * _C, _ZT)
    pltpu.sync_copy(acc.at[sl], out_hbm.at[cid, sl])


def _relu_add(dst, src):
    def erow(r, carry):
        for k in range(_D // _L):
            sl = pl.ds(k * _L, _L)
            dst[r, sl] = jnp.maximum(dst[r, sl] + src[r, sl], 0.0)
        return carry

    lax.fori_loop(0, _C, erow, 0)


def _sc_segsum_body(data_hbm, idx_hbm, out_hbm, idxv,
                    r0, r1, r2, r3, l0, l1, l2, l3, s0, s1, s2, s3, acc):
    cid = lax.axis_index("c")
    sid = lax.axis_index("s")
    wid = cid * _NS + sid
    base = wid * _EW
    rows = (r0, r1, r2, r3)
    lsem = (l0, l1, l2, l3)
    ssem = (s0, s1, s2, s3)
    pltpu.sync_copy(idx_hbm.at[wid], idxv)
    _zero_acc_stripe(acc, r0, sid)
    for b in range(2):
        pltpu.async_copy(data_hbm.at[pl.ds(base + b * _C, _C)],
                         rows[b], lsem[b])
    plsc.subcore_barrier()

    def outer(c2, carry):
        for b in range(4):
            c = c2 * 4 + b
            pltpu.make_async_copy(data_hbm.at[pl.ds(base + c * _C, _C)],
                                  rows[b], lsem[b]).wait()
            pltpu.async_copy(rows[b], acc.at[idxv.at[c]], ssem[b], add=True)
            b2 = (b + 2) % 4

            def _wait_prev(b2=b2):
                pltpu.make_async_copy(
                    rows[b2], acc.at[idxv.at[c - 2]], ssem[b2]).wait()

            def _issue(b2=b2):
                pltpu.async_copy(
                    data_hbm.at[pl.ds(base + (c + 2) * _C, _C)],
                    rows[b2], lsem[b2])

            if b < 2:
                pl.when(c2 > 0)(_wait_prev)
                _issue()
            else:
                _wait_prev()
                pl.when(c2 < _K // 4 - 1)(_issue)
        return carry

    lax.fori_loop(0, _K // 4, outer, 0)
    for b in (2, 3):
        pltpu.make_async_copy(
            rows[b], acc.at[idxv.at[_K - 4 + b]], ssem[b]).wait()
    plsc.subcore_barrier()
    _flush_acc_stripe(acc, out_hbm, cid, sid)


def _sc_segsum(data, idx3):
    return pl.kernel(
        _sc_segsum_body,
        out_type=jax.ShapeDtypeStruct((_NC, _NV, _D), jnp.float32),
        mesh=_mesh(),
        compiler_params=_params(),
        scratch_types=[
            pltpu.VMEM((_K, _C), jnp.int32),
            pltpu.VMEM((_C, _D), jnp.float32),
            pltpu.VMEM((_C, _D), jnp.float32),
            pltpu.VMEM((_C, _D), jnp.float32),
            pltpu.VMEM((_C, _D), jnp.float32),
            pltpu.SemaphoreType.DMA,
            pltpu.SemaphoreType.DMA,
            pltpu.SemaphoreType.DMA,
            pltpu.SemaphoreType.DMA,
            pltpu.SemaphoreType.DMA,
            pltpu.SemaphoreType.DMA,
            pltpu.SemaphoreType.DMA,
            pltpu.SemaphoreType.DMA,
            pltpu.VMEM_SHARED((_NV, _D), jnp.float32),
        ],
    )(data, idx3)


def _sc_grs_body(a_hbm, b_hbm, gidx_hbm, sidx_hbm, m_hbm, out_hbm,
                 gidxv, sidxv, g0, g1, g2, g3, bb0, bb1,
                 gs0, gs1, gs2, gs3, ms0, ms1, ms2, ms3,
                 ss0, ss1, ss2, ss3, bs0, bs1, acc):
    cid = lax.axis_index("c")
    sid = lax.axis_index("s")
    wid = cid * _NS + sid
    base = wid * _EW
    g = (g0, g1, g2, g3)
    bb = (bb0, bb1)
    gsem = (gs0, gs1, gs2, gs3)
    msem = (ms0, ms1, ms2, ms3)
    ssem = (ss0, ss1, ss2, ss3)
    bsem = (bs0, bs1)
    pltpu.sync_copy(gidx_hbm.at[wid], gidxv)
    pltpu.sync_copy(sidx_hbm.at[wid], sidxv)
    _zero_acc_stripe(acc, g0, sid)
    for b in range(2):
        pltpu.async_copy(a_hbm.at[gidxv.at[b]], g[b], gsem[b])
        pltpu.async_copy(b_hbm.at[pl.ds(base + b * _C, _C)], bb[b], bsem[b])
    plsc.subcore_barrier()

    def outer(c2, carry):
        for b in range(4):
            bi = b % 2
            c = c2 * 4 + b
            pltpu.make_async_copy(a_hbm.at[gidxv.at[c]], g[b],
                                  gsem[b]).wait()
            pltpu.make_async_copy(b_hbm.at[pl.ds(base + c * _C, _C)],
                                  bb[bi], bsem[bi]).wait()
            _relu_add(g[b], bb[bi])

            def _issue_b(bi=bi):
                pltpu.async_copy(b_hbm.at[pl.ds(base + (c + 2) * _C, _C)],
                                 bb[bi], bsem[bi])

            if b < 2:
                _issue_b()
            else:
                pl.when(c2 < _K // 4 - 1)(_issue_b)
            pltpu.async_copy(g[b], m_hbm.at[pl.ds(base + c * _C, _C)],
                             msem[b])
            pltpu.async_copy(g[b], acc.at[sidxv.at[c]], ssem[b], add=True)
            b2 = (b + 2) % 4

            def _wait_prev(b2=b2):
                pltpu.make_async_copy(
                    g[b2], m_hbm.at[pl.ds(base + (c - 2) * _C, _C)],
                    msem[b2]).wait()
                pltpu.make_async_copy(
                    g[b2], acc.at[sidxv.at[c - 2]], ssem[b2]).wait()

            def _issue_g(b2=b2):
                pltpu.async_copy(a_hbm.at[gidxv.at[c + 2]], g[b2], gsem[b2])

            if b < 2:
                pl.when(c2 > 0)(_wait_prev)
                _issue_g()
            else:
                _wait_prev()
                pl.when(c2 < _K // 4 - 1)(_issue_g)
        return carry

    lax.fori_loop(0, _K // 4, outer, 0)
    for b in (2, 3):
        c = _K - 4 + b
        pltpu.make_async_copy(
            g[b], m_hbm.at[pl.ds(base + c * _C, _C)], msem[b]).wait()
        pltpu.make_async_copy(g[b], acc.at[sidxv.at[c]], ssem[b]).wait()
    plsc.subcore_barrier()
    _flush_acc_stripe(acc, out_hbm, cid, sid)


def _sc_gather_relu_scatter(a, b, gidx3, sidx3):
    return pl.kernel(
        _sc_grs_body,
        out_type=(jax.ShapeDtypeStruct((_E, _D), jnp.float32),
                  jax.ShapeDtypeStruct((_NC, _NF, _D), jnp.float32)),
        mesh=_mesh(),
        compiler_params=_params(),
        scratch_types=(
            [pltpu.VMEM((_K, _C), jnp.int32)] * 2
            + [pltpu.VMEM((_C, _D), jnp.float32)] * 6
            + [pltpu.SemaphoreType.DMA] * 14
            + [pltpu.VMEM_SHARED((_NF, _D), jnp.float32)]
        ),
    )(a, b, gidx3, sidx3)


def _sc_gr_body(a_hbm, b_hbm, gidx_hbm, m_hbm,
                gidxv, g0, g1, g2, g3, bb0, bb1,
                gs0, gs1, gs2, gs3, ms0, ms1, ms2, ms3,
                bs0, bs1, av_sh):
    cid = lax.axis_index("c")
    sid = lax.axis_index("s")
    wid = cid * _NS + sid
    base = wid * _EW
    g = (g0, g1, g2, g3)
    bb = (bb0, bb1)
    gsem = (gs0, gs1, gs2, gs3)
    msem = (ms0, ms1, ms2, ms3)
    bsem = (bs0, bs1)
    pltpu.sync_copy(gidx_hbm.at[wid], gidxv)
    for j in range(_ZJ):
        sl = pl.ds(sid * _STR + j * _C, _C)
        pltpu.sync_copy(a_hbm.at[sl], av_sh.at[sl])
    sl = pl.ds(sid * _STR + _ZJ * _C, _ZT)
    pltpu.sync_copy(a_hbm.at[sl], av_sh.at[sl])
    for b in range(2):
        pltpu.async_copy(b_hbm.at[pl.ds(base + b * _C, _C)], bb[b], bsem[b])
    plsc.subcore_barrier()
    for b in range(2):
        pltpu.async_copy(av_sh.at[gidxv.at[b]], g[b], gsem[b])

    def outer(c2, carry):
        for b in range(4):
            bi = b % 2
            c = c2 * 4 + b
            pltpu.make_async_copy(av_sh.at[gidxv.at[c]], g[b],
                                  gsem[b]).wait()
            pltpu.make_async_copy(b_hbm.at[pl.ds(base + c * _C, _C)],
                                  bb[bi], bsem[bi]).wait()
            _relu_add(g[b], bb[bi])

            def _issue_b(bi=bi):
                pltpu.async_copy(b_hbm.at[pl.ds(base + (c + 2) * _C, _C)],
                                 bb[bi], bsem[bi])

            if b < 2:
                _issue_b()
            else:
                pl.when(c2 < _K // 4 - 1)(_issue_b)
            pltpu.async_copy(g[b], m_hbm.at[pl.ds(base + c * _C, _C)],
                             msem[b])
            b2 = (b + 2) % 4

            def _wait_prev(b2=b2):
                pltpu.make_async_copy(
                    g[b2], m_hbm.at[pl.ds(base + (c - 2) * _C, _C)],
                    msem[b2]).wait()

            def _issue_g(b2=b2):
                pltpu.async_copy(av_sh.at[gidxv.at[c + 2]], g[b2], gsem[b2])

            if b < 2:
                pl.when(c2 > 0)(_wait_prev)
                _issue_g()
            else:
                _wait_prev()
                pl.when(c2 < _K // 4 - 1)(_issue_g)
        return carry

    lax.fori_loop(0, _K // 4, outer, 0)
    for b in (2, 3):
        c = _K - 4 + b
        pltpu.make_async_copy(
            g[b], m_hbm.at[pl.ds(base + c * _C, _C)], msem[b]).wait()


def _sc_gather_relu(a, b, gidx3):
    return pl.kernel(
        _sc_gr_body,
        out_type=jax.ShapeDtypeStruct((_E, _D), jnp.float32),
        mesh=_mesh(),
        compiler_params=_params(),
        scratch_types=(
            [pltpu.VMEM((_K, _C), jnp.int32)]
            + [pltpu.VMEM((_C, _D), jnp.float32)] * 6
            + [pltpu.SemaphoreType.DMA] * 10
            + [pltpu.VMEM_SHARED((_NF, _D), jnp.float32)]
        ),
    )(a, b, gidx3)


_EB = 16000


def _tc_cat_body(x_ref, y_ref, w_ref, o_ref):
    h = jnp.concatenate((x_ref[...], y_ref[...]), axis=-1)
    o_ref[...] = jnp.dot(h, w_ref[...], preferred_element_type=jnp.float32)


def _tc_edge_cat(x, y, wcat, eb=_EB):
    return pl.pallas_call(
        _tc_cat_body,
        grid=(_E // eb,),
        in_specs=[
            pl.BlockSpec((eb, _D), lambda i: (i, 0)),
            pl.BlockSpec((eb, _D), lambda i: (i, 0)),
            pl.BlockSpec((2 * _D, _D), lambda i: (0, 0)),
        ],
        out_specs=pl.BlockSpec((eb, _D), lambda i: (i, 0)),
        out_shape=jax.ShapeDtypeStruct((_E, _D), jnp.float32),
    )(x, y, wcat)


_NB = 10000


def _tc_node_body(s0_ref, s1_ref, x_ref, w1_ref, w2_ref, b_ref,
                  a_ref, nx_ref):
    s = s0_ref[0] + s1_ref[0]
    a_ref[...] = (
        jnp.dot(x_ref[...], w1_ref[...], preferred_element_type=jnp.float32)
        + jnp.dot(s, w2_ref[...], preferred_element_type=jnp.float32)
        + b_ref[...])
    nx_ref[...] = s + x_ref[...]


def _tc_node_combo(sp, x, w1, w2, b):
    n = x.shape[0]
    return pl.pallas_call(
        _tc_node_body,
        grid=(n // _NB,),
        in_specs=[
            pl.BlockSpec((1, _NB, _D), lambda i: (0, i, 0)),
            pl.BlockSpec((1, _NB, _D), lambda i: (1, i, 0)),
            pl.BlockSpec((_NB, _D), lambda i: (i, 0)),
            pl.BlockSpec((_D, _D), lambda i: (0, 0)),
            pl.BlockSpec((_D, _D), lambda i: (0, 0)),
            pl.BlockSpec((1, _D), lambda i: (0, 0)),
        ],
        out_specs=[
            pl.BlockSpec((_NB, _D), lambda i: (i, 0)),
            pl.BlockSpec((_NB, _D), lambda i: (i, 0)),
        ],
        out_shape=[
            jax.ShapeDtypeStruct((n, _D), jnp.float32),
            jax.ShapeDtypeStruct((n, _D), jnp.float32),
        ],
    )(sp, sp, x, w1, w2, b)


def kernel(variable, factor, edge_attr, prev_m_f_to_v, v_to_f, f_to_v,
           W_v, b_v, W_f, b_f):
    gidx = v_to_f.reshape(_NW, _K, _C)
    sidx = f_to_v.reshape(_NW, _K, _C)
    Wv1, Wv2, Wv3 = W_v[:_D], W_v[_D:2 * _D], W_v[2 * _D:]
    Wf1, Wf2, Wf3 = W_f[:_D], W_f[_D:2 * _D], W_f[2 * _D:]

    s1p = _sc_segsum(prev_m_f_to_v, gidx)
    b1 = _tc_edge_cat(edge_attr, prev_m_f_to_v,
                      jnp.concatenate((Wv3, -Wv2), axis=0))
    a_v, new_variable = _tc_node_combo(s1p, variable, Wv1, Wv2,
                                       b_v.reshape(1, _D))
    m_v_to_f, s2p = _sc_gather_relu_scatter(a_v, b1, gidx, sidx)
    a_f, new_factor = _tc_node_combo(s2p, factor, Wf1, Wf2,
                                     b_f.reshape(1, _D))
    p2 = _tc_edge_cat(edge_attr, m_v_to_f,
                      jnp.concatenate((Wf3, -Wf2), axis=0))
    m_f_to_v = _sc_gather_relu(a_f, p2, sidx)
    return (m_f_to_v, new_factor, new_variable)
